# baseline (speedup 1.0000x reference)
import functools

import jax
import jax.numpy as jnp
from jax import lax
from jax.experimental import pallas as pl
from jax.experimental.pallas import tpu as pltpu
from jax.experimental.pallas import tpu_sc as plsc

B = 8
P = 2048
L = 16
INF = float("inf")

SCB = 2
BPC = SCB // 2
NW = 16 // BPC
CH = P // NW
RG = 8
NJ = P // L

TCB = B - SCB
BR = 256
NBLK = P // BR


def _sc_kernel(c1_hbm, c2_hbm, out_hbm,
               c1_v, c2_v, m2_v, rs_v, cmb_v, o_v, sh):
    c = lax.axis_index("c")
    s = lax.axis_index("s")
    b = c * BPC + s // NW
    q = s % NW

    pltpu.sync_copy(c1_hbm.at[b, :, pl.ds(q * CH, CH)], c1_v)
    pltpu.sync_copy(c2_hbm.at[b], c2_v)

    def pre2(k, _):
        m2_v[0, pl.ds(k * L, L)] = jnp.full((L,), INF, jnp.float32)
        return 0
    lax.fori_loop(0, NJ, pre2, 0)

    def outer(g, rowsum):
        pxv = c1_v[0, pl.ds(g * L, L)]
        pyv = c1_v[1, pl.ds(g * L, L)]
        pzv = c1_v[2, pl.ds(g * L, L)]
        for half in range(L // RG):
            px = [pxv[half * RG + r] for r in range(RG)]
            py = [pyv[half * RG + r] for r in range(RG)]
            pz = [pzv[half * RG + r] for r in range(RG)]

            def inner(j, carry):
                rm = list(carry)
                xv = c2_v[0, pl.ds(j * L, L)]
                yv = c2_v[1, pl.ds(j * L, L)]
                zv = c2_v[2, pl.ds(j * L, L)]
                u = []
                for r in range(RG):
                    dx = xv - px[r]
                    dy = yv - py[r]
                    dz = zv - pz[r]
                    t = dx * dx + dy * dy + dz * dz
                    rm[r] = jnp.minimum(rm[r], t)
                    u.append(t)
                while len(u) > 1:
                    u = [jnp.minimum(u[2 * i], u[2 * i + 1])
                         for i in range(len(u) // 2)]
                m2v = m2_v[0, pl.ds(j * L, L)]
                m2_v[0, pl.ds(j * L, L)] = jnp.minimum(m2v, u[0])
                return tuple(rm)

            init = (jnp.full((L,), INF, jnp.float32),) * RG
            rm = lax.fori_loop(0, NJ, inner, init)
            for r in range(RG):
                rowsum = rowsum + jnp.min(rm[r])
        return rowsum

    rowsum = lax.fori_loop(0, CH // L, outer, jnp.float32(0.0))

    rs_v[0, :] = jnp.full((L,), rowsum, jnp.float32)
    pltpu.sync_copy(m2_v, sh.at[s, :, pl.ds(0, P)])
    pltpu.sync_copy(rs_v, sh.at[s, :, pl.ds(P, L)])
    plsc.subcore_barrier()

    @pl.when(q == 0)
    def _combine():
        pltpu.sync_copy(sh.at[pl.ds(s, NW)], cmb_v)

        def red(k, acc):
            w = [cmb_v[r, 0, pl.ds(k * L, L)] for r in range(NW)]
            while len(w) > 1:
                w = [jnp.minimum(w[2 * i], w[2 * i + 1])
                     for i in range(len(w) // 2)]
            return acc + w[0]
        acc = lax.fori_loop(0, NJ, red, jnp.zeros((L,), jnp.float32))
        total2 = jnp.sum(acc)
        t1v = cmb_v[0, 0, pl.ds(P, L)]
        for r in range(1, NW):
            t1v = t1v + cmb_v[r, 0, pl.ds(P, L)]
        total1 = t1v[0]
        res = (total1 + total2) * jnp.float32(1.0 / P)
        o_v[0, :] = jnp.full((L,), res, jnp.float32)
        pltpu.sync_copy(o_v, out_hbm.at[b])


def _sc_cham(c1t, c2t):
    mesh = plsc.VectorSubcoreMesh(core_axis_name="c", subcore_axis_name="s")
    f = pl.kernel(
        _sc_kernel,
        mesh=mesh,
        compiler_params=pltpu.CompilerParams(needs_layout_passes=False),
        out_type=jax.ShapeDtypeStruct((SCB, 1, L), jnp.float32),
        scratch_types=[
            pltpu.VMEM((3, CH), jnp.float32),
            pltpu.VMEM((3, P), jnp.float32),
            pltpu.VMEM((1, P), jnp.float32),
            pltpu.VMEM((1, L), jnp.float32),
            pltpu.VMEM((NW, 1, P + L), jnp.float32),
            pltpu.VMEM((1, L), jnp.float32),
            pltpu.VMEM_SHARED((16, 1, P + L), jnp.float32),
        ],
    )
    return f(c1t, c2t)


def _tc_kernel(c1_ref, c1t_ref, c2_ref, c2t_ref, out_ref,
               n2_ref, n2c_ref, colmin_ref, rowsum_ref):
    j = pl.program_id(1)

    c2 = c2_ref[0]
    c2t = c2t_ref[0]
    a = c1_ref[0]
    at = c1t_ref[0]

    @pl.when(j == 0)
    def _init():
        n2_ref[...] = jnp.sum(c2 * c2, axis=0, keepdims=True)
        n2c_ref[...] = jnp.sum(c2t * c2t, axis=1, keepdims=True)
        colmin_ref[...] = jnp.full((1, P), jnp.inf, jnp.float32)
        rowsum_ref[...] = jnp.zeros((1, 128), jnp.float32)

    n1c = jnp.sum(a * a, axis=1, keepdims=True)
    n1r = jnp.sum(at * at, axis=0, keepdims=True)
    s2 = jnp.dot(a * jnp.float32(-2.0), c2,
                 preferred_element_type=jnp.float32,
                 precision=lax.Precision.HIGHEST)
    colmin_ref[...] = jnp.minimum(colmin_ref[...],
                                  jnp.min(s2 + n1c, axis=0, keepdims=True))
    v = jnp.dot(c2t * jnp.float32(-2.0), at,
                preferred_element_type=jnp.float32,
                precision=lax.Precision.HIGHEST)
    rowmin = jnp.min(v + n2c_ref[...], axis=0, keepdims=True)
    rowsum_ref[...] += jnp.full((1, 128),
                                jnp.sum(rowmin + n1r), jnp.float32)

    @pl.when(j == NBLK - 1)
    def _fin():
        total2 = jnp.sum(colmin_ref[...] + n2_ref[...])
        total1 = rowsum_ref[0, 0]
        out_ref[...] = jnp.full((1, 1, 128), (total1 + total2)
                                * jnp.float32(1.0 / P), jnp.float32)


def _tc_cham(c1p, c1tp, c2p, c2tp):
    return pl.pallas_call(
        _tc_kernel,
        grid=(TCB, NBLK),
        in_specs=[
            pl.BlockSpec((1, BR, 8), lambda b, j: (b, j, 0)),
            pl.BlockSpec((1, 8, BR), lambda b, j: (b, 0, j)),
            pl.BlockSpec((1, 8, P), lambda b, j: (b, 0, 0)),
            pl.BlockSpec((1, P, 8), lambda b, j: (b, 0, 0)),
        ],
        out_specs=pl.BlockSpec((1, 1, 128), lambda b, j: (b, 0, 0)),
        out_shape=jax.ShapeDtypeStruct((TCB, 1, 128), jnp.float32),
        scratch_shapes=[
            pltpu.VMEM((1, P), jnp.float32),
            pltpu.VMEM((P, 1), jnp.float32),
            pltpu.VMEM((1, P), jnp.float32),
            pltpu.VMEM((1, 128), jnp.float32),
        ],
    )(c1p, c1tp, c2p, c2tp)


@jax.jit
def _cham(c1t_sc, c2t_sc, c1p_tc, c1tp_tc, c2p_tc, c2tp_tc):
    out_sc = _sc_cham(c1t_sc, c2t_sc)
    out_tc = _tc_cham(c1p_tc, c1tp_tc, c2p_tc, c2tp_tc)
    return jnp.concatenate([out_sc[:, 0, 0], out_tc[:, 0, 0]])


def kernel(cloud1, cloud2):
    c1t = cloud1[:SCB].transpose(0, 2, 1)
    c2t = cloud2[:SCB].transpose(0, 2, 1)
    z1 = jnp.zeros((TCB, P, 5), jnp.float32)
    c1p = jnp.concatenate([cloud1[SCB:], z1], axis=2)
    c2tp = jnp.concatenate([cloud2[SCB:], z1], axis=2)
    z2 = jnp.zeros((TCB, 5, P), jnp.float32)
    c1tp = jnp.concatenate([cloud1[SCB:].transpose(0, 2, 1), z2], axis=1)
    c2p = jnp.concatenate([cloud2[SCB:].transpose(0, 2, 1), z2], axis=1)
    return _cham(c1t, c2t, c1p, c1tp, c2p, c2tp)


# speedup vs baseline: 1.8832x; 1.8832x over previous
import functools

import jax
import jax.numpy as jnp
from jax import lax
from jax.experimental import pallas as pl
from jax.experimental.pallas import tpu as pltpu
from jax.experimental.pallas import tpu_sc as plsc

B = 8
P = 2048
L = 16
INF = float("inf")

SCB = 2
BPC = SCB // 2
NW = 16 // BPC
CH = P // NW
RG = 8
NJ = P // L

TCB = B - SCB
BR = 512
NBLK = P // BR


def _sc_kernel(c1_hbm, c2_hbm, out_hbm,
               c1_v, c2_v, m2_v, rs_v, cmb_v, o_v, sh):
    c = lax.axis_index("c")
    s = lax.axis_index("s")
    b = c * BPC + s // NW
    q = s % NW

    pltpu.sync_copy(c1_hbm.at[b, :, pl.ds(q * CH, CH)], c1_v)
    pltpu.sync_copy(c2_hbm.at[b], c2_v)

    def pre2(k, _):
        m2_v[0, pl.ds(k * L, L)] = jnp.full((L,), INF, jnp.float32)
        return 0
    lax.fori_loop(0, NJ, pre2, 0)

    def outer(g, rowsum):
        pxv = c1_v[0, pl.ds(g * L, L)]
        pyv = c1_v[1, pl.ds(g * L, L)]
        pzv = c1_v[2, pl.ds(g * L, L)]
        for half in range(L // RG):
            px = [pxv[half * RG + r] for r in range(RG)]
            py = [pyv[half * RG + r] for r in range(RG)]
            pz = [pzv[half * RG + r] for r in range(RG)]

            def inner(j, carry):
                rm = list(carry)
                xv = c2_v[0, pl.ds(j * L, L)]
                yv = c2_v[1, pl.ds(j * L, L)]
                zv = c2_v[2, pl.ds(j * L, L)]
                u = []
                for r in range(RG):
                    dx = xv - px[r]
                    dy = yv - py[r]
                    dz = zv - pz[r]
                    t = dx * dx + dy * dy + dz * dz
                    rm[r] = jnp.minimum(rm[r], t)
                    u.append(t)
                while len(u) > 1:
                    u = [jnp.minimum(u[2 * i], u[2 * i + 1])
                         for i in range(len(u) // 2)]
                m2v = m2_v[0, pl.ds(j * L, L)]
                m2_v[0, pl.ds(j * L, L)] = jnp.minimum(m2v, u[0])
                return tuple(rm)

            init = (jnp.full((L,), INF, jnp.float32),) * RG
            rm = lax.fori_loop(0, NJ, inner, init)
            for r in range(RG):
                rowsum = rowsum + jnp.min(rm[r])
        return rowsum

    rowsum = lax.fori_loop(0, CH // L, outer, jnp.float32(0.0))

    rs_v[0, :] = jnp.full((L,), rowsum, jnp.float32)
    pltpu.sync_copy(m2_v, sh.at[s, :, pl.ds(0, P)])
    pltpu.sync_copy(rs_v, sh.at[s, :, pl.ds(P, L)])
    plsc.subcore_barrier()

    @pl.when(q == 0)
    def _combine():
        pltpu.sync_copy(sh.at[pl.ds(s, NW)], cmb_v)

        def red(k, acc):
            w = [cmb_v[r, 0, pl.ds(k * L, L)] for r in range(NW)]
            while len(w) > 1:
                w = [jnp.minimum(w[2 * i], w[2 * i + 1])
                     for i in range(len(w) // 2)]
            return acc + w[0]
        acc = lax.fori_loop(0, NJ, red, jnp.zeros((L,), jnp.float32))
        total2 = jnp.sum(acc)
        t1v = cmb_v[0, 0, pl.ds(P, L)]
        for r in range(1, NW):
            t1v = t1v + cmb_v[r, 0, pl.ds(P, L)]
        total1 = t1v[0]
        res = (total1 + total2) * jnp.float32(1.0 / P)
        o_v[0, :] = jnp.full((L,), res, jnp.float32)
        pltpu.sync_copy(o_v, out_hbm.at[b])


def _sc_cham(c1t, c2t):
    mesh = plsc.VectorSubcoreMesh(core_axis_name="c", subcore_axis_name="s")
    f = pl.kernel(
        _sc_kernel,
        mesh=mesh,
        compiler_params=pltpu.CompilerParams(needs_layout_passes=False),
        out_type=jax.ShapeDtypeStruct((SCB, 1, L), jnp.float32),
        scratch_types=[
            pltpu.VMEM((3, CH), jnp.float32),
            pltpu.VMEM((3, P), jnp.float32),
            pltpu.VMEM((1, P), jnp.float32),
            pltpu.VMEM((1, L), jnp.float32),
            pltpu.VMEM((NW, 1, P + L), jnp.float32),
            pltpu.VMEM((1, L), jnp.float32),
            pltpu.VMEM_SHARED((16, 1, P + L), jnp.float32),
        ],
    )
    return f(c1t, c2t)


def _tc_kernel(c1_ref, c2_ref, out_ref, colmin_ref, rowsum_ref):
    j = pl.program_id(1)

    @pl.when(j == 0)
    def _init():
        colmin_ref[...] = jnp.full((1, P), jnp.inf, jnp.float32)
        rowsum_ref[...] = jnp.zeros((1, 128), jnp.float32)

    d = jnp.dot(c1_ref[0], c2_ref[0],
                preferred_element_type=jnp.float32,
                precision=lax.Precision.HIGHEST)
    colmin_ref[...] = jnp.minimum(colmin_ref[...],
                                  jnp.min(d, axis=0, keepdims=True))
    rowmin = jnp.min(d, axis=1, keepdims=True)
    rowsum_ref[...] += jnp.full((1, 128), jnp.sum(rowmin), jnp.float32)

    @pl.when(j == NBLK - 1)
    def _fin():
        total = rowsum_ref[0, 0] + jnp.sum(colmin_ref[...])
        out_ref[...] = jnp.full((1, 1, 128),
                                total * jnp.float32(1.0 / P), jnp.float32)


def _tc_cham(c1a, c2a):
    return pl.pallas_call(
        _tc_kernel,
        grid=(TCB, NBLK),
        in_specs=[
            pl.BlockSpec((1, BR, 8), lambda b, j: (b, j, 0)),
            pl.BlockSpec((1, 8, P), lambda b, j: (b, 0, 0)),
        ],
        out_specs=pl.BlockSpec((1, 1, 128), lambda b, j: (b, 0, 0)),
        out_shape=jax.ShapeDtypeStruct((TCB, 1, 128), jnp.float32),
        scratch_shapes=[
            pltpu.VMEM((1, P), jnp.float32),
            pltpu.VMEM((1, 128), jnp.float32),
        ],
    )(c1a, c2a)


@jax.jit
def _cham(c1t_sc, c2t_sc, c1a_tc, c2a_tc):
    out_sc = _sc_cham(c1t_sc, c2t_sc)
    out_tc = _tc_cham(c1a_tc, c2a_tc)
    return jnp.concatenate([out_sc[:, 0, 0], out_tc[:, 0, 0]])


def kernel(cloud1, cloud2):
    c1t = cloud1[:SCB].transpose(0, 2, 1)
    c2t = cloud2[:SCB].transpose(0, 2, 1)
    c1 = cloud1[SCB:]
    c2 = cloud2[SCB:]
    n1 = jnp.sum(c1 * c1, axis=2, keepdims=True)
    n2 = jnp.sum(c2 * c2, axis=2, keepdims=True)
    ones = jnp.ones((TCB, P, 1), jnp.float32)
    zeros = jnp.zeros((TCB, P, 3), jnp.float32)
    c1a = jnp.concatenate([c1, n1, ones, zeros], axis=2)
    c2a = jnp.concatenate([-2.0 * c2, ones, n2, zeros],
                          axis=2).transpose(0, 2, 1)
    return _cham(c1t, c2t, c1a, c2a)


# speedup vs baseline: 1.9333x; 1.0266x over previous
import functools

import jax
import jax.numpy as jnp
from jax import lax
from jax.experimental import pallas as pl
from jax.experimental.pallas import tpu as pltpu
from jax.experimental.pallas import tpu_sc as plsc

B = 8
P = 2048
L = 16
INF = float("inf")

SCB = 2
BPC = SCB // 2
NW = 16 // BPC
CH = P // NW
RG = 8
NJ = P // L

TCB = B - SCB
BR = 512
NBLK = P // BR


def _sc_kernel(c1_hbm, c2_hbm, out_hbm,
               c1_v, c2_v, m2_v, cmb_v, o_v, sh):
    c = lax.axis_index("c")
    s = lax.axis_index("s")
    b = c * BPC + s // NW
    q = s % NW

    pltpu.sync_copy(c1_hbm.at[b, :, pl.ds(q * CH, CH)], c1_v)
    pltpu.sync_copy(c2_hbm.at[b], c2_v)

    def pre2(k, _):
        m2_v[0, pl.ds(k * L, L)] = jnp.full((L,), INF, jnp.float32)
        return 0
    lax.fori_loop(0, NJ, pre2, 0)

    def outer(g, rowsum):
        pxv = c1_v[0, pl.ds(g * L, L)]
        pyv = c1_v[1, pl.ds(g * L, L)]
        pzv = c1_v[2, pl.ds(g * L, L)]
        for half in range(L // RG):
            px = [pxv[half * RG + r] for r in range(RG)]
            py = [pyv[half * RG + r] for r in range(RG)]
            pz = [pzv[half * RG + r] for r in range(RG)]

            def inner(j, carry):
                rm = list(carry)
                xv = c2_v[0, pl.ds(j * L, L)]
                yv = c2_v[1, pl.ds(j * L, L)]
                zv = c2_v[2, pl.ds(j * L, L)]
                u = []
                for r in range(RG):
                    dx = xv - px[r]
                    dy = yv - py[r]
                    dz = zv - pz[r]
                    t = dx * dx + dy * dy + dz * dz
                    rm[r] = jnp.minimum(rm[r], t)
                    u.append(t)
                while len(u) > 1:
                    u = [jnp.minimum(u[2 * i], u[2 * i + 1])
                         for i in range(len(u) // 2)]
                m2v = m2_v[0, pl.ds(j * L, L)]
                m2_v[0, pl.ds(j * L, L)] = jnp.minimum(m2v, u[0])
                return tuple(rm)

            init = (jnp.full((L,), INF, jnp.float32),) * RG
            rm = lax.fori_loop(0, NJ, inner, init)
            for r in range(RG):
                rowsum = rowsum + jnp.min(rm[r])
        return rowsum

    rowsum = lax.fori_loop(0, CH // L, outer, jnp.float32(0.0))

    m2_v[0, pl.ds(P, L)] = jnp.full((L,), rowsum, jnp.float32)
    pltpu.sync_copy(m2_v, sh.at[s])
    plsc.subcore_barrier()

    @pl.when(q == 0)
    def _combine():
        pltpu.sync_copy(sh.at[pl.ds(s, NW)], cmb_v)

        def red(k, acc):
            w = [cmb_v[r, 0, pl.ds(k * L, L)] for r in range(NW)]
            while len(w) > 1:
                w = [jnp.minimum(w[2 * i], w[2 * i + 1])
                     for i in range(len(w) // 2)]
            return acc + w[0]
        acc = lax.fori_loop(0, NJ, red, jnp.zeros((L,), jnp.float32))
        total2 = jnp.sum(acc)
        t1v = cmb_v[0, 0, pl.ds(P, L)]
        for r in range(1, NW):
            t1v = t1v + cmb_v[r, 0, pl.ds(P, L)]
        total1 = t1v[0]
        res = (total1 + total2) * jnp.float32(1.0 / P)
        o_v[0, :] = jnp.full((L,), res, jnp.float32)
        pltpu.sync_copy(o_v, out_hbm.at[b])


def _sc_cham(c1t, c2t):
    mesh = plsc.VectorSubcoreMesh(core_axis_name="c", subcore_axis_name="s")
    f = pl.kernel(
        _sc_kernel,
        mesh=mesh,
        compiler_params=pltpu.CompilerParams(needs_layout_passes=False),
        out_type=jax.ShapeDtypeStruct((SCB, 1, L), jnp.float32),
        scratch_types=[
            pltpu.VMEM((3, CH), jnp.float32),
            pltpu.VMEM((3, P), jnp.float32),
            pltpu.VMEM((1, P + L), jnp.float32),
            pltpu.VMEM((NW, 1, P + L), jnp.float32),
            pltpu.VMEM((1, L), jnp.float32),
            pltpu.VMEM_SHARED((16, 1, P + L), jnp.float32),
        ],
    )
    return f(c1t, c2t)


def _tc_kernel(c1_ref, c2_ref, out_ref, colmin_ref, rowsum_ref):
    j = pl.program_id(1)

    @pl.when(j == 0)
    def _init():
        colmin_ref[...] = jnp.full((1, P), jnp.inf, jnp.float32)
        rowsum_ref[...] = jnp.zeros((1, 128), jnp.float32)

    d = jnp.dot(c1_ref[0], c2_ref[0],
                preferred_element_type=jnp.float32,
                precision=lax.Precision.HIGHEST)
    colmin_ref[...] = jnp.minimum(colmin_ref[...],
                                  jnp.min(d, axis=0, keepdims=True))
    rowmin = jnp.min(d, axis=1, keepdims=True)
    rowsum_ref[...] += jnp.full((1, 128), jnp.sum(rowmin), jnp.float32)

    @pl.when(j == NBLK - 1)
    def _fin():
        total = rowsum_ref[0, 0] + jnp.sum(colmin_ref[...])
        out_ref[...] = jnp.full((1, 1, 128),
                                total * jnp.float32(1.0 / P), jnp.float32)


def _tc_cham(c1a, c2a):
    return pl.pallas_call(
        _tc_kernel,
        grid=(TCB, NBLK),
        in_specs=[
            pl.BlockSpec((1, BR, 8), lambda b, j: (b, j, 0)),
            pl.BlockSpec((1, 8, P), lambda b, j: (b, 0, 0)),
        ],
        out_specs=pl.BlockSpec((1, 1, 128), lambda b, j: (b, 0, 0)),
        out_shape=jax.ShapeDtypeStruct((TCB, 1, 128), jnp.float32),
        scratch_shapes=[
            pltpu.VMEM((1, P), jnp.float32),
            pltpu.VMEM((1, 128), jnp.float32),
        ],
    )(c1a, c2a)


@jax.jit
def _cham(c1t_sc, c2t_sc, c1a_tc, c2a_tc):
    out_sc = _sc_cham(c1t_sc, c2t_sc)
    out_tc = _tc_cham(c1a_tc, c2a_tc)
    return jnp.concatenate([out_sc[:, 0, 0], out_tc[:, 0, 0]])


def kernel(cloud1, cloud2):
    c1t = cloud1[:SCB].transpose(0, 2, 1)
    c2t = cloud2[:SCB].transpose(0, 2, 1)
    c1 = cloud1[SCB:]
    c2 = cloud2[SCB:]
    n1 = jnp.sum(c1 * c1, axis=2, keepdims=True)
    n2 = jnp.sum(c2 * c2, axis=2, keepdims=True)
    ones = jnp.ones((TCB, P, 1), jnp.float32)
    zeros = jnp.zeros((TCB, P, 3), jnp.float32)
    c1a = jnp.concatenate([c1, n1, ones, zeros], axis=2)
    c2a = jnp.concatenate([-2.0 * c2, ones, n2, zeros],
                          axis=2).transpose(0, 2, 1)
    return _cham(c1t, c2t, c1a, c2a)


# speedup vs baseline: 1.9631x; 1.0154x over previous
import functools

import jax
import jax.numpy as jnp
from jax import lax
from jax.experimental import pallas as pl
from jax.experimental.pallas import tpu as pltpu
from jax.experimental.pallas import tpu_sc as plsc

B = 8
P = 2048
L = 16
INF = float("inf")

SCB = 2
BPC = SCB // 2
NW = 16 // BPC
CH = P // NW
RG = 8
NJ = P // L

TCB = B - SCB
BR = 1024
NBLK = P // BR


def _sc_kernel(c1_hbm, c2_hbm, out_hbm,
               c1_v, c2_v, m2_v, cmb_v, o_v, sh):
    c = lax.axis_index("c")
    s = lax.axis_index("s")
    b = c * BPC + s // NW
    q = s % NW

    pltpu.sync_copy(c1_hbm.at[b, :, pl.ds(q * CH, CH)], c1_v)
    pltpu.sync_copy(c2_hbm.at[b], c2_v)

    def pre2(k, _):
        m2_v[0, pl.ds(k * L, L)] = jnp.full((L,), INF, jnp.float32)
        return 0
    lax.fori_loop(0, NJ, pre2, 0)

    def outer(g, rowsum):
        pxv = c1_v[0, pl.ds(g * L, L)]
        pyv = c1_v[1, pl.ds(g * L, L)]
        pzv = c1_v[2, pl.ds(g * L, L)]
        for half in range(L // RG):
            px = [pxv[half * RG + r] for r in range(RG)]
            py = [pyv[half * RG + r] for r in range(RG)]
            pz = [pzv[half * RG + r] for r in range(RG)]

            def inner(j, carry):
                rm = list(carry)
                xv = c2_v[0, pl.ds(j * L, L)]
                yv = c2_v[1, pl.ds(j * L, L)]
                zv = c2_v[2, pl.ds(j * L, L)]
                u = []
                for r in range(RG):
                    dx = xv - px[r]
                    dy = yv - py[r]
                    dz = zv - pz[r]
                    t = dx * dx + dy * dy + dz * dz
                    rm[r] = jnp.minimum(rm[r], t)
                    u.append(t)
                while len(u) > 1:
                    u = [jnp.minimum(u[2 * i], u[2 * i + 1])
                         for i in range(len(u) // 2)]
                m2v = m2_v[0, pl.ds(j * L, L)]
                m2_v[0, pl.ds(j * L, L)] = jnp.minimum(m2v, u[0])
                return tuple(rm)

            init = (jnp.full((L,), INF, jnp.float32),) * RG
            rm = lax.fori_loop(0, NJ, inner, init)
            for r in range(RG):
                rowsum = rowsum + jnp.min(rm[r])
        return rowsum

    rowsum = lax.fori_loop(0, CH // L, outer, jnp.float32(0.0))

    m2_v[0, pl.ds(P, L)] = jnp.full((L,), rowsum, jnp.float32)
    pltpu.sync_copy(m2_v, sh.at[s])
    plsc.subcore_barrier()

    @pl.when(q == 0)
    def _combine():
        pltpu.sync_copy(sh.at[pl.ds(s, NW)], cmb_v)

        def red(k, acc):
            w = [cmb_v[r, 0, pl.ds(k * L, L)] for r in range(NW)]
            while len(w) > 1:
                w = [jnp.minimum(w[2 * i], w[2 * i + 1])
                     for i in range(len(w) // 2)]
            return acc + w[0]
        acc = lax.fori_loop(0, NJ, red, jnp.zeros((L,), jnp.float32))
        total2 = jnp.sum(acc)
        t1v = cmb_v[0, 0, pl.ds(P, L)]
        for r in range(1, NW):
            t1v = t1v + cmb_v[r, 0, pl.ds(P, L)]
        total1 = t1v[0]
        res = (total1 + total2) * jnp.float32(1.0 / P)
        o_v[0, :] = jnp.full((L,), res, jnp.float32)
        pltpu.sync_copy(o_v, out_hbm.at[b])


def _sc_cham(c1t, c2t):
    mesh = plsc.VectorSubcoreMesh(core_axis_name="c", subcore_axis_name="s")
    f = pl.kernel(
        _sc_kernel,
        mesh=mesh,
        compiler_params=pltpu.CompilerParams(needs_layout_passes=False),
        out_type=jax.ShapeDtypeStruct((SCB, 1, L), jnp.float32),
        scratch_types=[
            pltpu.VMEM((3, CH), jnp.float32),
            pltpu.VMEM((3, P), jnp.float32),
            pltpu.VMEM((1, P + L), jnp.float32),
            pltpu.VMEM((NW, 1, P + L), jnp.float32),
            pltpu.VMEM((1, L), jnp.float32),
            pltpu.VMEM_SHARED((16, 1, P + L), jnp.float32),
        ],
    )
    return f(c1t, c2t)


def _tc_kernel(c1_ref, c2_ref, out_ref, colmin_ref, rowsum_ref):
    j = pl.program_id(1)

    @pl.when(j == 0)
    def _init():
        colmin_ref[...] = jnp.full((1, P), jnp.inf, jnp.float32)
        rowsum_ref[...] = jnp.zeros((1, 128), jnp.float32)

    d = jnp.dot(c1_ref[0], c2_ref[0],
                preferred_element_type=jnp.float32,
                precision=lax.Precision.HIGHEST)
    colmin_ref[...] = jnp.minimum(colmin_ref[...],
                                  jnp.min(d, axis=0, keepdims=True))
    rowmin = jnp.min(d, axis=1, keepdims=True)
    rowsum_ref[...] += jnp.full((1, 128), jnp.sum(rowmin), jnp.float32)

    @pl.when(j == NBLK - 1)
    def _fin():
        total = rowsum_ref[0, 0] + jnp.sum(colmin_ref[...])
        out_ref[...] = jnp.full((1, 1, 128),
                                total * jnp.float32(1.0 / P), jnp.float32)


def _tc_cham(c1a, c2a):
    return pl.pallas_call(
        _tc_kernel,
        grid=(TCB, NBLK),
        in_specs=[
            pl.BlockSpec((1, BR, 8), lambda b, j: (b, j, 0)),
            pl.BlockSpec((1, 8, P), lambda b, j: (b, 0, 0)),
        ],
        out_specs=pl.BlockSpec((1, 1, 128), lambda b, j: (b, 0, 0)),
        out_shape=jax.ShapeDtypeStruct((TCB, 1, 128), jnp.float32),
        scratch_shapes=[
            pltpu.VMEM((1, P), jnp.float32),
            pltpu.VMEM((1, 128), jnp.float32),
        ],
    )(c1a, c2a)


@jax.jit
def _cham(c1t_sc, c2t_sc, c1a_tc, c2a_tc):
    out_sc = _sc_cham(c1t_sc, c2t_sc)
    out_tc = _tc_cham(c1a_tc, c2a_tc)
    return jnp.concatenate([out_sc[:, 0, 0], out_tc[:, 0, 0]])


def kernel(cloud1, cloud2):
    c1t = cloud1[:SCB].transpose(0, 2, 1)
    c2t = cloud2[:SCB].transpose(0, 2, 1)
    c1 = cloud1[SCB:]
    c2 = cloud2[SCB:]
    n1 = jnp.sum(c1 * c1, axis=2, keepdims=True)
    n2 = jnp.sum(c2 * c2, axis=2, keepdims=True)
    ones = jnp.ones((TCB, P, 1), jnp.float32)
    zeros = jnp.zeros((TCB, P, 3), jnp.float32)
    c1a = jnp.concatenate([c1, n1, ones, zeros], axis=2)
    c2a = jnp.concatenate([-2.0 * c2, ones, n2, zeros],
                          axis=2).transpose(0, 2, 1)
    return _cham(c1t, c2t, c1a, c2a)
